# Initial kernel scaffold; baseline (speedup 1.0000x reference)
import jax
import jax.numpy as jnp
from jax.experimental import pallas as pl


def kernel(x_cats, x_conts, tables, bn_cont_g, bn_cont_b, W1, b1, g1, be1, W2, b2, g2, be2, W3, b3):
    raise NotImplementedError("write your pallas kernel here")


# speedup vs baseline: 7.6987x; 7.6987x over previous
import jax
import jax.numpy as jnp
from jax import lax
from jax.experimental import pallas as pl
from jax.experimental.pallas import tpu as pltpu
from jax.experimental.pallas import tpu_sc as plsc

B = 16384
F = 26
V = 100000
D = 16
R = B * F
NC = 2
NS = 16
NW = NC * NS
RPW = R // NW
SPW = RPW // 128
SJ = 8
CR = SJ * 128
NCH = SPW // SJ

EPS = 1e-5
INV = (1.0 + EPS) ** -0.5

H1 = 512
H2 = 256
OUT = 2
EMBW = F * D
N_CONTS = 13
BT = 2048


def _gather_body(idx_hbm, tab_hbm, out_hbm, idx_v, rows_v, sem):
    wid = lax.axis_index("s") * NC + lax.axis_index("c")
    for c in range(NCH):
        sbase = wid * SPW + c * SJ
        rbase = wid * RPW + c * CR
        pltpu.sync_copy(idx_hbm.at[pl.ds(sbase, SJ)], idx_v)
        copies = [
            pltpu.async_copy(tab_hbm.at[idx_v.at[j]],
                             rows_v.at[pl.ds(j * 128, 128)], sem)
            for j in range(SJ)
        ]
        for cp in copies:
            cp.wait()
        pltpu.sync_copy(rows_v, out_hbm.at[pl.ds(rbase, CR)])


def _sc_gather(flat_idx_2d, tab_flat):
    return pl.kernel(
        _gather_body,
        out_type=jax.ShapeDtypeStruct((R, D), jnp.float32),
        scratch_types=[
            pltpu.VMEM((SJ, 128), jnp.int32),
            pltpu.VMEM((CR, D), jnp.float32),
            pltpu.SemaphoreType.DMA,
        ],
        mesh=plsc.VectorSubcoreMesh(core_axis_name="c", subcore_axis_name="s"),
        compiler_params=pltpu.CompilerParams(use_tc_tiling_on_sc=False),
    )(flat_idx_2d, tab_flat)


def _mlp_body(x_ref, xc_ref, cg_ref, cb_ref, w1a_ref, w1b_ref, b1_ref,
              g1_ref, be1_ref, w2_ref, b2_ref, g2_ref, be2_ref,
              w3_ref, b3_ref, out_ref):
    xc = cg_ref[...] * (xc_ref[...] * INV) + cb_ref[...]
    z = jnp.dot(x_ref[...], w1a_ref[...], preferred_element_type=jnp.float32)
    z = z + jnp.dot(xc, w1b_ref[...], preferred_element_type=jnp.float32)
    z = z + b1_ref[...]
    h = jnp.maximum(z, 0.0)
    h = g1_ref[...] * (h * INV) + be1_ref[...]
    z = jnp.dot(h, w2_ref[...], preferred_element_type=jnp.float32) + b2_ref[...]
    h = jnp.maximum(z, 0.0)
    h = g2_ref[...] * (h * INV) + be2_ref[...]
    out_ref[...] = jnp.dot(h, w3_ref[...], preferred_element_type=jnp.float32) + b3_ref[...]


def _mlp(x, x_conts, cg, cb, W1a, W1b, b1, g1, be1, W2, b2, g2, be2, W3, b3):
    grid = (B // BT,)
    row = lambda i: (i, 0)
    rep = lambda i: (0, 0)
    return pl.pallas_call(
        _mlp_body,
        grid=grid,
        in_specs=[
            pl.BlockSpec((BT, EMBW), row),
            pl.BlockSpec((BT, N_CONTS), row),
            pl.BlockSpec((1, N_CONTS), rep),
            pl.BlockSpec((1, N_CONTS), rep),
            pl.BlockSpec((EMBW, H1), rep),
            pl.BlockSpec((N_CONTS, H1), rep),
            pl.BlockSpec((1, H1), rep),
            pl.BlockSpec((1, H1), rep),
            pl.BlockSpec((1, H1), rep),
            pl.BlockSpec((H1, H2), rep),
            pl.BlockSpec((1, H2), rep),
            pl.BlockSpec((1, H2), rep),
            pl.BlockSpec((1, H2), rep),
            pl.BlockSpec((H2, OUT), rep),
            pl.BlockSpec((1, OUT), rep),
        ],
        out_specs=pl.BlockSpec((BT, OUT), row),
        out_shape=jax.ShapeDtypeStruct((B, OUT), jnp.float32),
    )(x, x_conts, cg, cb, W1a, W1b, b1, g1, be1, W2, b2, g2, be2, W3, b3)


def kernel(x_cats, x_conts, tables, bn_cont_g, bn_cont_b,
           W1, b1, g1, be1, W2, b2, g2, be2, W3, b3):
    xi = x_cats.astype(jnp.int32)
    offs = (jnp.arange(F, dtype=jnp.int32) * V)[None, :]
    flat_idx = (xi + offs).reshape(R // 128, 128)
    tab_flat = tables.reshape(F * V, D)
    emb = _sc_gather(flat_idx, tab_flat)
    x = emb.reshape(B, EMBW)
    W1a = W1[:EMBW]
    W1b = W1[EMBW:]
    return _mlp(x, x_conts, bn_cont_g[None, :], bn_cont_b[None, :],
                W1a, W1b, b1[None, :], g1[None, :], be1[None, :],
                W2, b2[None, :], g2[None, :], be2[None, :],
                W3, b3[None, :])


# speedup vs baseline: 7.8260x; 1.0165x over previous
import jax
import jax.numpy as jnp
from jax import lax
from jax.experimental import pallas as pl
from jax.experimental.pallas import tpu as pltpu
from jax.experimental.pallas import tpu_sc as plsc

B = 16384
F = 26
V = 100000
D = 16
R = B * F
NC = 2
NS = 16
NW = NC * NS
BPW = B // NW
RPW = R // NW
NCH = 8
CR = RPW // NCH

EPS = 1e-5
INV = (1.0 + EPS) ** -0.5

H1 = 512
H2 = 256
OUT = 2
EMBW = F * D
N_CONTS = 13
BT = 2048


def _gather_body(idx_hbm, tab_hbm, out_hbm, i0, i1, r0, r1, gs0, gs1, ws0, ws1):
    wid = lax.axis_index("s") * NC + lax.axis_index("c")
    f0 = wid * RPW
    idx_v = [i0, i1]
    rows_v = [r0, r1]
    gsem = [gs0, gs1]
    wsem = [ws0, ws1]
    gd = [None, None]
    wd = [None, None]

    def fire(c):
        b = c & 1
        pltpu.sync_copy(idx_hbm.at[pl.ds(f0 + c * CR, CR)], idx_v[b])
        gd[b] = pltpu.async_copy(tab_hbm.at[idx_v[b]], rows_v[b], gsem[b])

    fire(0)
    for c in range(NCH):
        b = c & 1
        nb = (c + 1) & 1
        if c + 1 < NCH:
            if wd[nb] is not None:
                wd[nb].wait()
            fire(c + 1)
        gd[b].wait()
        wd[b] = pltpu.make_async_copy(rows_v[b],
                                      out_hbm.at[pl.ds(f0 + c * CR, CR)],
                                      wsem[b])
        wd[b].start()
    wd[0].wait()
    wd[1].wait()


def _sc_gather(flat_idx, tab_flat):
    return pl.kernel(
        _gather_body,
        out_type=jax.ShapeDtypeStruct((R, D), jnp.float32),
        scratch_types=[
            pltpu.VMEM((CR,), jnp.int32),
            pltpu.VMEM((CR,), jnp.int32),
            pltpu.VMEM((CR, D), jnp.float32),
            pltpu.VMEM((CR, D), jnp.float32),
            pltpu.SemaphoreType.DMA,
            pltpu.SemaphoreType.DMA,
            pltpu.SemaphoreType.DMA,
            pltpu.SemaphoreType.DMA,
        ],
        mesh=plsc.VectorSubcoreMesh(core_axis_name="c", subcore_axis_name="s"),
        compiler_params=pltpu.CompilerParams(use_tc_tiling_on_sc=False),
    )(flat_idx, tab_flat)


def _mlp_body(x_ref, xc_ref, cg_ref, cb_ref, w1a_ref, w1b_ref, b1_ref,
              g1_ref, be1_ref, w2_ref, b2_ref, g2_ref, be2_ref,
              w3_ref, b3_ref, out_ref):
    xc = cg_ref[...] * (xc_ref[...] * INV) + cb_ref[...]
    z = jnp.dot(x_ref[...], w1a_ref[...], preferred_element_type=jnp.float32)
    z = z + jnp.dot(xc, w1b_ref[...], preferred_element_type=jnp.float32)
    z = z + b1_ref[...]
    h = jnp.maximum(z, 0.0)
    h = g1_ref[...] * (h * INV) + be1_ref[...]
    z = jnp.dot(h, w2_ref[...], preferred_element_type=jnp.float32) + b2_ref[...]
    h = jnp.maximum(z, 0.0)
    h = g2_ref[...] * (h * INV) + be2_ref[...]
    out_ref[...] = jnp.dot(h, w3_ref[...], preferred_element_type=jnp.float32) + b3_ref[...]


def _mlp(x, x_conts, cg, cb, W1a, W1b, b1, g1, be1, W2, b2, g2, be2, W3, b3):
    grid = (B // BT,)
    row = lambda i: (i, 0)
    rep = lambda i: (0, 0)
    return pl.pallas_call(
        _mlp_body,
        grid=grid,
        in_specs=[
            pl.BlockSpec((BT, EMBW), row),
            pl.BlockSpec((BT, N_CONTS), row),
            pl.BlockSpec((1, N_CONTS), rep),
            pl.BlockSpec((1, N_CONTS), rep),
            pl.BlockSpec((EMBW, H1), rep),
            pl.BlockSpec((N_CONTS, H1), rep),
            pl.BlockSpec((1, H1), rep),
            pl.BlockSpec((1, H1), rep),
            pl.BlockSpec((1, H1), rep),
            pl.BlockSpec((H1, H2), rep),
            pl.BlockSpec((1, H2), rep),
            pl.BlockSpec((1, H2), rep),
            pl.BlockSpec((1, H2), rep),
            pl.BlockSpec((H2, OUT), rep),
            pl.BlockSpec((1, OUT), rep),
        ],
        out_specs=pl.BlockSpec((BT, OUT), row),
        out_shape=jax.ShapeDtypeStruct((B, OUT), jnp.float32),
    )(x, x_conts, cg, cb, W1a, W1b, b1, g1, be1, W2, b2, g2, be2, W3, b3)


def kernel(x_cats, x_conts, tables, bn_cont_g, bn_cont_b,
           W1, b1, g1, be1, W2, b2, g2, be2, W3, b3):
    xi = x_cats.astype(jnp.int32)
    offs = (jnp.arange(F, dtype=jnp.int32) * V)[None, :]
    flat_idx = (xi + offs).reshape(R)
    tab_flat = tables.reshape(F * V, D)
    x = _sc_gather(flat_idx, tab_flat).reshape(B, EMBW)
    W1a = W1[:EMBW]
    W1b = W1[EMBW:]
    return _mlp(x, x_conts, bn_cont_g[None, :], bn_cont_b[None, :],
                W1a, W1b, b1[None, :], g1[None, :], be1[None, :],
                W2, b2[None, :], g2[None, :], be2[None, :],
                W3, b3[None, :])
